# Initial kernel scaffold; baseline (speedup 1.0000x reference)
import jax
import jax.numpy as jnp
from jax.experimental import pallas as pl


def kernel(inputs, edge_index, adj_values, epsilon):
    raise NotImplementedError("write your pallas kernel here")


# speedup vs baseline: 2.5563x; 2.5563x over previous
import functools

import jax
import jax.numpy as jnp
from jax import lax
from jax.experimental import pallas as pl
from jax.experimental.pallas import tpu as pltpu
from jax.experimental.pallas import tpu_sc as plsc

B = 8
N = 10000
C = 2
E = 160000
D = 256
H = 128
NTILE = 16
EPT = C * E // NTILE
K = 80
NCHUNK = EPT // K
NP = 10240
RPT = NP // NTILE
RC = 80


def _build_sc_call():
    mesh = plsc.VectorSubcoreMesh(core_axis_name="c", subcore_axis_name="s")

    @functools.partial(
        pl.kernel,
        mesh=mesh,
        out_type=jax.ShapeDtypeStruct((B, 2, N, H), jnp.float32),
        scratch_types=[
            pltpu.VMEM_SHARED((NP, H), jnp.float32),
            pltpu.VMEM((K,), jnp.int32),
            pltpu.VMEM((K,), jnp.int32),
            pltpu.VMEM((K,), jnp.float32),
            pltpu.VMEM((K, H), jnp.float32),
            pltpu.VMEM((RC, H), jnp.float32),
            pltpu.VMEM((16,), jnp.float32),
            pltpu.SemaphoreType.DMA,
        ],
    )
    def gin_sc(xt_hbm, rows_hbm, cols_hbm, vals_hbm, esv_hbm, out_hbm,
               acc, idxv, rowv, valv, buf, cbuf, epsv, sem):
        h = lax.axis_index("c")
        s = lax.axis_index("s")
        pltpu.sync_copy(esv_hbm, epsv)
        esplat = epsv[pl.ds(0, 16)]
        tile_eoff = s * EPT
        r0 = s * RPT
        nrch = jnp.minimum(RPT, jnp.maximum(0, N - r0)) // RC

        def per_graph(b, carry):
            base = (b * 2 + h) * N

            def init_chunk(rc, c2):
                ro = r0 + rc * RC
                pltpu.sync_copy(xt_hbm.at[pl.ds(base + ro, RC)], cbuf)

                def scale_row(i, c3):
                    for j in range(H // 16):
                        sl = pl.ds(j * 16, 16)
                        cbuf[i, sl] = esplat * cbuf[i, sl]
                    return c3

                lax.fori_loop(0, RC, scale_row, 0)
                pltpu.sync_copy(cbuf, acc.at[pl.ds(ro, RC)])
                return c2

            lax.fori_loop(0, nrch, init_chunk, 0)
            plsc.subcore_barrier()

            def edge_chunk(g, c2):
                off = b * (C * E) + tile_eoff + g * K
                pltpu.sync_copy(cols_hbm.at[pl.ds(off, K)], idxv)
                pltpu.sync_copy(rows_hbm.at[pl.ds(off, K)], rowv)
                pltpu.sync_copy(vals_hbm.at[pl.ds(off, K)], valv)
                for j in range(K // 16):
                    sl = pl.ds(j * 16, 16)
                    idxv[sl] = idxv[sl] + base
                pltpu.async_copy(xt_hbm.at[idxv], buf, sem).wait()

                def scale_group(g2, c3):
                    vvec = valv[pl.ds(g2 * 16, 16)]

                    def scale_edge(i2, c4):
                        vsplat = vvec[jnp.full((16,), i2, jnp.int32)]
                        i = g2 * 16 + i2
                        for j in range(H // 16):
                            sl = pl.ds(j * 16, 16)
                            buf[i, sl] = vsplat * buf[i, sl]
                        return c4

                    lax.fori_loop(0, 16, scale_edge, 0)
                    return c3

                lax.fori_loop(0, K // 16, scale_group, 0)
                pltpu.sync_copy(buf, acc.at[rowv], add=True)
                return c2

            lax.fori_loop(0, NCHUNK, edge_chunk, 0)
            plsc.subcore_barrier()

            def out_chunk(rc, c2):
                ro = r0 + rc * RC
                pltpu.sync_copy(acc.at[pl.ds(ro, RC)], cbuf)
                pltpu.sync_copy(cbuf, out_hbm.at[b, h, pl.ds(ro, RC)])
                return c2

            lax.fori_loop(0, nrch, out_chunk, 0)
            plsc.subcore_barrier()
            return carry

        lax.fori_loop(0, B, per_graph, 0)

    return gin_sc


_GIN_SC = _build_sc_call()


@jax.jit
def _gin(inputs, edge_index, adj_values, epsilon):
    ei = edge_index.astype(jnp.int32)
    rows = ei[:, :, 0, :].reshape(B * C * E)
    cols = ei[:, :, 1, :].reshape(B * C * E)
    vals = adj_values.astype(jnp.float32).reshape(B * C * E)
    xt = (inputs.reshape(B, N, 2, H)
          .transpose(0, 2, 1, 3)
          .reshape(B * 2 * N, H))
    esv = jnp.full((16,), jnp.sum(epsilon), jnp.float32)
    out = _GIN_SC(xt, rows, cols, vals, esv)
    return out.transpose(0, 2, 1, 3).reshape(B, N, D)


def kernel(inputs, edge_index, adj_values, epsilon):
    return _gin(inputs, edge_index, adj_values, epsilon)


# speedup vs baseline: 7.0887x; 2.7730x over previous
import functools

import jax
import jax.numpy as jnp
from jax import lax
from jax.experimental import pallas as pl
from jax.experimental.pallas import tpu as pltpu
from jax.experimental.pallas import tpu_sc as plsc

B = 8
N = 10000
C = 2
E = 160000
D = 256
H = 128
NTILE = 16
EPT = C * E // NTILE
K = 80
NCHUNK = EPT // K
BLK = 2000
NBLK = EPT // BLK
CPB = BLK // K
NP = 10240
RPT = NP // NTILE
RC = 80


def _build_sc_call():
    mesh = plsc.VectorSubcoreMesh(core_axis_name="c", subcore_axis_name="s")

    @functools.partial(
        pl.kernel,
        mesh=mesh,
        out_type=jax.ShapeDtypeStruct((B, 2, N, H), jnp.float32),
        scratch_types=[
            pltpu.VMEM_SHARED((NP, H), jnp.float32),
            pltpu.VMEM((2 * BLK,), jnp.int32),
            pltpu.VMEM((2 * BLK,), jnp.int32),
            pltpu.VMEM((2 * BLK,), jnp.float32),
            pltpu.VMEM((K,), jnp.int32),
            pltpu.VMEM((K,), jnp.int32),
            pltpu.VMEM((K,), jnp.int32),
            pltpu.VMEM((K, H), jnp.float32),
            pltpu.VMEM((K, H), jnp.float32),
            pltpu.VMEM((16,), jnp.float32),
            pltpu.SemaphoreType.DMA,
            pltpu.SemaphoreType.DMA,
            pltpu.SemaphoreType.DMA,
            pltpu.SemaphoreType.DMA,
            pltpu.SemaphoreType.DMA,
        ],
    )
    def gin_sc(xt_hbm, rows_hbm, cols_hbm, vals_hbm, esv_hbm, out_hbm,
               acc, rows_st, cols_st, vals_st, idxv0, idxv1, rowv,
               buf0, buf1, epsv,
               sem_g0, sem_g1, sem_r, sem_c, sem_v):
        h = lax.axis_index("c")
        s = lax.axis_index("s")
        pltpu.sync_copy(esv_hbm, epsv)
        esplat = epsv[pl.ds(0, 16)]
        r0 = s * RPT
        nrch = jnp.minimum(RPT, jnp.maximum(0, N - r0)) // RC
        idxvs = (idxv0, idxv1)
        bufs = (buf0, buf1)
        sems = (sem_g0, sem_g1)

        def per_graph(b, carry):
            hbase = (b * 2 + h) * N
            eoff = b * (C * E) + s * EPT

            def stage_start(blk):
                po = pl.multiple_of((blk % 2) * BLK, 8)
                off = eoff + blk * BLK
                pltpu.async_copy(rows_hbm.at[pl.ds(off, BLK)],
                                 rows_st.at[pl.ds(po, BLK)], sem_r)
                pltpu.async_copy(cols_hbm.at[pl.ds(off, BLK)],
                                 cols_st.at[pl.ds(po, BLK)], sem_c)
                pltpu.async_copy(vals_hbm.at[pl.ds(off, BLK)],
                                 vals_st.at[pl.ds(po, BLK)], sem_v)

            def stage_wait(blk):
                po = pl.multiple_of((blk % 2) * BLK, 8)
                off = eoff + blk * BLK
                pltpu.make_async_copy(rows_hbm.at[pl.ds(off, BLK)],
                                      rows_st.at[pl.ds(po, BLK)], sem_r).wait()
                pltpu.make_async_copy(cols_hbm.at[pl.ds(off, BLK)],
                                      cols_st.at[pl.ds(po, BLK)], sem_c).wait()
                pltpu.make_async_copy(vals_hbm.at[pl.ds(off, BLK)],
                                      vals_st.at[pl.ds(po, BLK)], sem_v).wait()

            def build_idx(g, dstv):
                p = (g // CPB) % 2
                gb = g % CPB
                for j in range(K // 16):
                    src = pl.multiple_of(p * BLK + gb * K + j * 16, 16)
                    dstv[pl.ds(j * 16, 16)] = cols_st[pl.ds(src, 16)] + hbase

            stage_start(0)

            def init_chunk(rc, c2):
                ro = r0 + rc * RC
                pltpu.sync_copy(xt_hbm.at[pl.ds(hbase + ro, RC)], buf0)

                @plsc.parallel_loop(0, RC, unroll=2)
                def _(i):
                    for j in range(H // 16):
                        sl = pl.ds(j * 16, 16)
                        buf0[i, sl] = esplat * buf0[i, sl]

                pltpu.sync_copy(buf0, acc.at[pl.ds(ro, RC)])
                return c2

            lax.fori_loop(0, nrch, init_chunk, 0)

            stage_wait(0)
            stage_start(1)
            plsc.subcore_barrier()

            build_idx(0, idxvs[0])
            pltpu.async_copy(xt_hbm.at[idxvs[0]], bufs[0], sems[0])

            def scale_chunk(g, buf):
                p = (g // CPB) % 2
                gb = g % CPB

                @plsc.parallel_loop(0, K, unroll=4)
                def _(i):
                    ga = i // 16
                    i2 = i - ga * 16
                    voff = pl.multiple_of(p * BLK + gb * K + ga * 16, 16)
                    vvec = vals_st[pl.ds(voff, 16)]
                    vsplat = vvec[jnp.full((16,), i2, jnp.int32)]
                    for j in range(H // 16):
                        sl = pl.ds(j * 16, 16)
                        buf[i, sl] = vsplat * buf[i, sl]

            def outer(g2, c2):
                for par in range(2):
                    g = g2 * 2 + par
                    nxt = par ^ 1

                    @pl.when((g % CPB == 0) & (g >= CPB)
                             & (g <= (NBLK - 2) * CPB))
                    def _():
                        stage_start(g // CPB + 1)

                    @pl.when(g + 1 < NCHUNK)
                    def _():
                        @pl.when((g + 1) % CPB == 0)
                        def _():
                            stage_wait((g + 1) // CPB)

                        build_idx(g + 1, idxvs[nxt])
                        pltpu.async_copy(xt_hbm.at[idxvs[nxt]], bufs[nxt],
                                         sems[nxt])

                    pltpu.make_async_copy(xt_hbm.at[idxvs[par]], bufs[par],
                                          sems[par]).wait()
                    scale_chunk(g, bufs[par])
                    p = (g // CPB) % 2
                    gb = g % CPB
                    for j in range(K // 16):
                        src = pl.multiple_of(p * BLK + gb * K + j * 16, 16)
                        rowv[pl.ds(j * 16, 16)] = rows_st[pl.ds(src, 16)]
                    pltpu.sync_copy(bufs[par], acc.at[rowv], add=True)
                return c2

            lax.fori_loop(0, NCHUNK // 2, outer, 0)
            plsc.subcore_barrier()

            def out_chunk(rc, c2):
                ro = r0 + rc * RC
                pltpu.sync_copy(acc.at[pl.ds(ro, RC)], buf0)
                pltpu.sync_copy(buf0, out_hbm.at[b, h, pl.ds(ro, RC)])
                return c2

            lax.fori_loop(0, nrch, out_chunk, 0)
            plsc.subcore_barrier()
            return carry

        lax.fori_loop(0, B, per_graph, 0)

    return gin_sc


_GIN_SC = _build_sc_call()


@jax.jit
def _gin(inputs, edge_index, adj_values, epsilon):
    ei = edge_index.astype(jnp.int32)
    rows = ei[:, :, 0, :].reshape(B * C * E)
    cols = ei[:, :, 1, :].reshape(B * C * E)
    vals = adj_values.astype(jnp.float32).reshape(B * C * E)
    xt = (inputs.reshape(B, N, 2, H)
          .transpose(0, 2, 1, 3)
          .reshape(B * 2 * N, H))
    esv = jnp.full((16,), jnp.sum(epsilon), jnp.float32)
    out = _GIN_SC(xt, rows, cols, vals, esv)
    return out.transpose(0, 2, 1, 3).reshape(B, N, D)


def kernel(inputs, edge_index, adj_values, epsilon):
    return _gin(inputs, edge_index, adj_values, epsilon)


# speedup vs baseline: 7.2080x; 1.0168x over previous
import functools

import jax
import jax.numpy as jnp
from jax import lax
from jax.experimental import pallas as pl
from jax.experimental.pallas import tpu as pltpu
from jax.experimental.pallas import tpu_sc as plsc

B = 8
N = 10000
C = 2
E = 160000
D = 256
H = 128
NTILE = 16
EPT = C * E // NTILE
K = 80
NCHUNK = EPT // K
BLK = 2000
NBLK = EPT // BLK
CPB = BLK // K
NP = 10240
RPT = NP // NTILE
RC = 80


def _build_sc_call():
    mesh = plsc.VectorSubcoreMesh(core_axis_name="c", subcore_axis_name="s")

    @functools.partial(
        pl.kernel,
        mesh=mesh,
        out_type=jax.ShapeDtypeStruct((B, 2, N, H), jnp.float32),
        scratch_types=[
            pltpu.VMEM_SHARED((NP, H), jnp.float32),
            pltpu.VMEM((2 * BLK,), jnp.int32),
            pltpu.VMEM((2 * BLK,), jnp.int32),
            pltpu.VMEM((2 * BLK,), jnp.float32),
            pltpu.VMEM((K,), jnp.int32),
            pltpu.VMEM((K,), jnp.int32),
            pltpu.VMEM((K,), jnp.int32),
            pltpu.VMEM((K,), jnp.int32),
            pltpu.VMEM((K, H), jnp.float32),
            pltpu.VMEM((K, H), jnp.float32),
            pltpu.VMEM((16,), jnp.float32),
            pltpu.SemaphoreType.DMA,
            pltpu.SemaphoreType.DMA,
            pltpu.SemaphoreType.DMA,
            pltpu.SemaphoreType.DMA,
            pltpu.SemaphoreType.DMA,
            pltpu.SemaphoreType.DMA,
        ],
    )
    def gin_sc(xt_hbm, rows_hbm, cols_hbm, vals_hbm, esv_hbm, out_hbm,
               acc, rows_st, cols_st, vals_st, idxv0, idxv1, rowv0, rowv1,
               buf0, buf1, epsv,
               sem_g0, sem_g1, sem_s, sem_r, sem_c, sem_v):
        h = lax.axis_index("c")
        s = lax.axis_index("s")
        pltpu.sync_copy(esv_hbm, epsv)
        esplat = epsv[pl.ds(0, 16)]
        r0 = s * RPT
        nrch = jnp.minimum(RPT, jnp.maximum(0, N - r0)) // RC
        idxvs = (idxv0, idxv1)
        rowvs = (rowv0, rowv1)
        bufs = (buf0, buf1)
        sems = (sem_g0, sem_g1)

        def per_graph(b, carry):
            hbase = (b * 2 + h) * N
            eoff = b * (C * E) + s * EPT

            def stage_start(blk):
                po = pl.multiple_of((blk % 2) * BLK, 8)
                off = eoff + blk * BLK
                pltpu.async_copy(rows_hbm.at[pl.ds(off, BLK)],
                                 rows_st.at[pl.ds(po, BLK)], sem_r)
                pltpu.async_copy(cols_hbm.at[pl.ds(off, BLK)],
                                 cols_st.at[pl.ds(po, BLK)], sem_c)
                pltpu.async_copy(vals_hbm.at[pl.ds(off, BLK)],
                                 vals_st.at[pl.ds(po, BLK)], sem_v)

            def stage_wait(blk):
                po = pl.multiple_of((blk % 2) * BLK, 8)
                off = eoff + blk * BLK
                pltpu.make_async_copy(rows_hbm.at[pl.ds(off, BLK)],
                                      rows_st.at[pl.ds(po, BLK)], sem_r).wait()
                pltpu.make_async_copy(cols_hbm.at[pl.ds(off, BLK)],
                                      cols_st.at[pl.ds(po, BLK)], sem_c).wait()
                pltpu.make_async_copy(vals_hbm.at[pl.ds(off, BLK)],
                                      vals_st.at[pl.ds(po, BLK)], sem_v).wait()

            def build_idx(g, dstv):
                p = (g // CPB) % 2
                gb = g % CPB
                for j in range(K // 16):
                    src = pl.multiple_of(p * BLK + gb * K + j * 16, 16)
                    dstv[pl.ds(j * 16, 16)] = cols_st[pl.ds(src, 16)] + hbase

            stage_start(0)

            def init_chunk(rc, c2):
                ro = r0 + rc * RC
                pltpu.sync_copy(xt_hbm.at[pl.ds(hbase + ro, RC)], buf0)

                @plsc.parallel_loop(0, RC, unroll=2)
                def _(i):
                    for j in range(H // 16):
                        sl = pl.ds(j * 16, 16)
                        buf0[i, sl] = esplat * buf0[i, sl]

                pltpu.sync_copy(buf0, acc.at[pl.ds(ro, RC)])
                return c2

            lax.fori_loop(0, nrch, init_chunk, 0)

            stage_wait(0)
            stage_start(1)
            plsc.subcore_barrier()

            build_idx(0, idxvs[0])
            pltpu.async_copy(xt_hbm.at[idxvs[0]], bufs[0], sems[0])

            def scale_chunk(g, buf):
                p = (g // CPB) % 2
                gb = g % CPB

                @plsc.parallel_loop(0, K, unroll=4)
                def _(i):
                    ga = i // 16
                    i2 = i - ga * 16
                    voff = pl.multiple_of(p * BLK + gb * K + ga * 16, 16)
                    vvec = vals_st[pl.ds(voff, 16)]
                    vsplat = vvec[jnp.full((16,), i2, jnp.int32)]
                    for j in range(H // 16):
                        sl = pl.ds(j * 16, 16)
                        buf[i, sl] = vsplat * buf[i, sl]

            def outer(g2, c2):
                for par in range(2):
                    g = g2 * 2 + par
                    nxt = par ^ 1

                    @pl.when((g % CPB == 0) & (g >= CPB)
                             & (g <= (NBLK - 2) * CPB))
                    def _():
                        stage_start(g // CPB + 1)

                    @pl.when(g + 1 < NCHUNK)
                    def _():
                        @pl.when(g >= 1)
                        def _():
                            pltpu.make_async_copy(
                                bufs[nxt], acc.at[rowvs[nxt]], sem_s).wait()

                        @pl.when((g + 1) % CPB == 0)
                        def _():
                            stage_wait((g + 1) // CPB)

                        build_idx(g + 1, idxvs[nxt])
                        pltpu.async_copy(xt_hbm.at[idxvs[nxt]], bufs[nxt],
                                         sems[nxt])

                    p = (g // CPB) % 2
                    gb = g % CPB
                    for j in range(K // 16):
                        src = pl.multiple_of(p * BLK + gb * K + j * 16, 16)
                        rowvs[par][pl.ds(j * 16, 16)] = rows_st[pl.ds(src, 16)]
                    pltpu.make_async_copy(xt_hbm.at[idxvs[par]], bufs[par],
                                          sems[par]).wait()
                    scale_chunk(g, bufs[par])
                    pltpu.async_copy(bufs[par], acc.at[rowvs[par]], sem_s,
                                     add=True)
                return c2

            lax.fori_loop(0, NCHUNK // 2, outer, 0)
            pltpu.make_async_copy(bufs[0], acc.at[rowvs[0]], sem_s).wait()
            pltpu.make_async_copy(bufs[1], acc.at[rowvs[1]], sem_s).wait()
            plsc.subcore_barrier()

            def out_chunk(rc, c2):
                ro = r0 + rc * RC
                pltpu.sync_copy(acc.at[pl.ds(ro, RC)], buf0)
                pltpu.sync_copy(buf0, out_hbm.at[b, h, pl.ds(ro, RC)])
                return c2

            lax.fori_loop(0, nrch, out_chunk, 0)
            plsc.subcore_barrier()
            return carry

        lax.fori_loop(0, B, per_graph, 0)

    return gin_sc


_GIN_SC = _build_sc_call()


@jax.jit
def _gin(inputs, edge_index, adj_values, epsilon):
    ei = edge_index.astype(jnp.int32)
    rows = ei[:, :, 0, :].reshape(B * C * E)
    cols = ei[:, :, 1, :].reshape(B * C * E)
    vals = adj_values.astype(jnp.float32).reshape(B * C * E)
    xt = (inputs.reshape(B, N, 2, H)
          .transpose(0, 2, 1, 3)
          .reshape(B * 2 * N, H))
    esv = jnp.full((16,), jnp.sum(epsilon), jnp.float32)
    out = _GIN_SC(xt, rows, cols, vals, esv)
    return out.transpose(0, 2, 1, 3).reshape(B, N, D)


def kernel(inputs, edge_index, adj_values, epsilon):
    return _gin(inputs, edge_index, adj_values, epsilon)


# speedup vs baseline: 8.0818x; 1.1212x over previous
import functools

import jax
import jax.numpy as jnp
from jax import lax
from jax.experimental import pallas as pl
from jax.experimental.pallas import tpu as pltpu
from jax.experimental.pallas import tpu_sc as plsc

B = 8
N = 10000
C = 2
E = 160000
D = 256
H = 128
NTILE = 16
EPT = C * E // NTILE
K = 80
NCHUNK = EPT // K
PD = 3
BLK = 2000
NBLK = EPT // BLK
CPB = BLK // K
NP = 10240
RPT = NP // NTILE
RC = 80


def _build_sc_call():
    mesh = plsc.VectorSubcoreMesh(core_axis_name="c", subcore_axis_name="s")

    @functools.partial(
        pl.kernel,
        mesh=mesh,
        out_type=jax.ShapeDtypeStruct((B, 2, N, H), jnp.float32),
        scratch_types=[
            pltpu.VMEM_SHARED((NP, H), jnp.float32),
            pltpu.VMEM((2 * BLK,), jnp.int32),
            pltpu.VMEM((2 * BLK,), jnp.int32),
            pltpu.VMEM((2 * BLK,), jnp.float32),
            pltpu.VMEM((K,), jnp.int32),
            pltpu.VMEM((K,), jnp.int32),
            pltpu.VMEM((K,), jnp.int32),
            pltpu.VMEM((K,), jnp.int32),
            pltpu.VMEM((K,), jnp.int32),
            pltpu.VMEM((K,), jnp.int32),
            pltpu.VMEM((K, H), jnp.float32),
            pltpu.VMEM((K, H), jnp.float32),
            pltpu.VMEM((K, H), jnp.float32),
            pltpu.VMEM((16,), jnp.float32),
            pltpu.SemaphoreType.DMA,
            pltpu.SemaphoreType.DMA,
            pltpu.SemaphoreType.DMA,
            pltpu.SemaphoreType.DMA,
            pltpu.SemaphoreType.DMA,
            pltpu.SemaphoreType.DMA,
            pltpu.SemaphoreType.DMA,
        ],
    )
    def gin_sc(xt_hbm, rows_hbm, cols_hbm, vals_hbm, esv_hbm, out_hbm,
               acc, rows_st, cols_st, vals_st,
               idxv0, idxv1, idxv2, rowv0, rowv1, rowv2,
               buf0, buf1, buf2, epsv,
               sem_g0, sem_g1, sem_g2, sem_s, sem_r, sem_c, sem_v):
        h = lax.axis_index("c")
        s = lax.axis_index("s")
        pltpu.sync_copy(esv_hbm, epsv)
        esplat = epsv[pl.ds(0, 16)]
        r0 = s * RPT
        nrch = jnp.minimum(RPT, jnp.maximum(0, N - r0)) // RC
        idxvs = (idxv0, idxv1, idxv2)
        rowvs = (rowv0, rowv1, rowv2)
        bufs = (buf0, buf1, buf2)
        sems = (sem_g0, sem_g1, sem_g2)

        def per_graph(b, carry):
            hbase = (b * 2 + h) * N
            eoff = b * (C * E) + s * EPT

            def stage_start(blk):
                po = pl.multiple_of((blk % 2) * BLK, 8)
                off = eoff + blk * BLK
                pltpu.async_copy(rows_hbm.at[pl.ds(off, BLK)],
                                 rows_st.at[pl.ds(po, BLK)], sem_r)
                pltpu.async_copy(cols_hbm.at[pl.ds(off, BLK)],
                                 cols_st.at[pl.ds(po, BLK)], sem_c)
                pltpu.async_copy(vals_hbm.at[pl.ds(off, BLK)],
                                 vals_st.at[pl.ds(po, BLK)], sem_v)

            def stage_wait(blk):
                po = pl.multiple_of((blk % 2) * BLK, 8)
                off = eoff + blk * BLK
                pltpu.make_async_copy(rows_hbm.at[pl.ds(off, BLK)],
                                      rows_st.at[pl.ds(po, BLK)], sem_r).wait()
                pltpu.make_async_copy(cols_hbm.at[pl.ds(off, BLK)],
                                      cols_st.at[pl.ds(po, BLK)], sem_c).wait()
                pltpu.make_async_copy(vals_hbm.at[pl.ds(off, BLK)],
                                      vals_st.at[pl.ds(po, BLK)], sem_v).wait()

            def build_idx(g, dstv):
                p = (g // CPB) % 2
                gb = g % CPB
                for j in range(K // 16):
                    src = pl.multiple_of(p * BLK + gb * K + j * 16, 16)
                    dstv[pl.ds(j * 16, 16)] = cols_st[pl.ds(src, 16)] + hbase

            stage_start(0)

            def init_chunk(rc, c2):
                ro = r0 + rc * RC
                pltpu.sync_copy(xt_hbm.at[pl.ds(hbase + ro, RC)], buf0)

                @plsc.parallel_loop(0, RC, unroll=2)
                def _(i):
                    for j in range(H // 16):
                        sl = pl.ds(j * 16, 16)
                        buf0[i, sl] = esplat * buf0[i, sl]

                pltpu.sync_copy(buf0, acc.at[pl.ds(ro, RC)])
                return c2

            lax.fori_loop(0, nrch, init_chunk, 0)

            stage_wait(0)
            stage_start(1)
            plsc.subcore_barrier()

            def scale_chunk(g, buf):
                p = (g // CPB) % 2
                gb = g % CPB

                @plsc.parallel_loop(0, K, unroll=4)
                def _(i):
                    ga = i // 16
                    i2 = i - ga * 16
                    voff = pl.multiple_of(p * BLK + gb * K + ga * 16, 16)
                    vvec = vals_st[pl.ds(voff, 16)]
                    vsplat = vvec[jnp.full((16,), i2, jnp.int32)]
                    for j in range(H // 16):
                        sl = pl.ds(j * 16, 16)
                        buf[i, sl] = vsplat * buf[i, sl]

            def process(g, par):
                pb = (par + 2) % PD

                @pl.when((g % CPB == 0) & (g >= CPB)
                         & (g <= (NBLK - 2) * CPB))
                def _():
                    stage_start(g // CPB + 1)

                @pl.when(g + 2 < NCHUNK)
                def _():
                    @pl.when(g >= 1)
                    def _():
                        pltpu.make_async_copy(
                            bufs[pb], acc.at[rowvs[pb]], sem_s).wait()

                    @pl.when((g + 2) % CPB == 0)
                    def _():
                        stage_wait((g + 2) // CPB)

                    build_idx(g + 2, idxvs[pb])
                    pltpu.async_copy(xt_hbm.at[idxvs[pb]], bufs[pb],
                                     sems[pb])

                p = (g // CPB) % 2
                gb = g % CPB
                for j in range(K // 16):
                    src = pl.multiple_of(p * BLK + gb * K + j * 16, 16)
                    rowvs[par][pl.ds(j * 16, 16)] = rows_st[pl.ds(src, 16)]
                pltpu.make_async_copy(xt_hbm.at[idxvs[par]], bufs[par],
                                      sems[par]).wait()
                scale_chunk(g, bufs[par])
                pltpu.async_copy(bufs[par], acc.at[rowvs[par]], sem_s,
                                 add=True)

            build_idx(0, idxvs[0])
            pltpu.async_copy(xt_hbm.at[idxvs[0]], bufs[0], sems[0])
            build_idx(1, idxvs[1])
            pltpu.async_copy(xt_hbm.at[idxvs[1]], bufs[1], sems[1])

            def outer(g3, c2):
                for par in range(PD):
                    process(g3 * PD + par, par)
                return c2

            lax.fori_loop(0, (NCHUNK - 1) // PD, outer, 0)
            process(jnp.int32(NCHUNK - 1), (NCHUNK - 1) % PD)

            for q in (NCHUNK - 3, NCHUNK - 2, NCHUNK - 1):
                pltpu.make_async_copy(bufs[q % PD], acc.at[rowvs[q % PD]],
                                      sem_s).wait()
            plsc.subcore_barrier()

            def out_chunk(rc, c2):
                ro = r0 + rc * RC
                pltpu.sync_copy(acc.at[pl.ds(ro, RC)], buf0)
                pltpu.sync_copy(buf0, out_hbm.at[b, h, pl.ds(ro, RC)])
                return c2

            lax.fori_loop(0, nrch, out_chunk, 0)
            plsc.subcore_barrier()
            return carry

        lax.fori_loop(0, B, per_graph, 0)

    return gin_sc


_GIN_SC = _build_sc_call()


@jax.jit
def _gin(inputs, edge_index, adj_values, epsilon):
    ei = edge_index.astype(jnp.int32)
    rows = ei[:, :, 0, :].reshape(B * C * E)
    cols = ei[:, :, 1, :].reshape(B * C * E)
    vals = adj_values.astype(jnp.float32).reshape(B * C * E)
    xt = (inputs.reshape(B, N, 2, H)
          .transpose(0, 2, 1, 3)
          .reshape(B * 2 * N, H))
    esv = jnp.full((16,), jnp.sum(epsilon), jnp.float32)
    out = _GIN_SC(xt, rows, cols, vals, esv)
    return out.transpose(0, 2, 1, 3).reshape(B, N, D)


def kernel(inputs, edge_index, adj_values, epsilon):
    return _gin(inputs, edge_index, adj_values, epsilon)
